# baseline (speedup 1.0000x reference)
import functools

import jax
import jax.numpy as jnp
from jax import lax
from jax.experimental import pallas as pl
from jax.experimental.pallas import tpu as pltpu
from jax.experimental.pallas import tpu_sc as plsc

_VOCAB = 1_000_000
_EMB_DIM = 64
_BATCH = 16384
_HIST_LEN = 50

_NC = 2
_NS = 16
_NW = _NC * _NS
_L = 16

_N_PER_W = _BATCH // _NW
_CHUNK = 128
_JBLK = _N_PER_W // _CHUNK
_NCH = _HIST_LEN * _JBLK

_NBUF = 4
_LEAD = 2


def _emb_body(xt_hbm, w_hbm, out_hbm, idx_v, rows_v, t_v, *sems):
    gsems, wsems = sems[:_NBUF], sems[_NBUF:]
    wid = lax.axis_index("s") * _NC + lax.axis_index("c")
    col0 = wid * _N_PER_W
    pltpu.sync_copy(xt_hbm.at[:, pl.ds(col0, _N_PER_W)], idx_v)

    lane = lax.iota(jnp.int32, _L)
    riota = [lane + n0 * _L for n0 in range(_CHUNK // _L)]

    def chunk_hj(c):
        return c % _HIST_LEN, c // _HIST_LEN

    def gather(c, b):
        h, j = chunk_hj(c)
        pltpu.async_copy(w_hbm.at[idx_v.at[h, pl.ds(j * _CHUNK, _CHUNK)]],
                         rows_v.at[b], gsems[b])

    def wait_gather(c, b):
        h, j = chunk_hj(c)
        pltpu.make_async_copy(
            w_hbm.at[idx_v.at[h, pl.ds(j * _CHUNK, _CHUNK)]],
            rows_v.at[b], gsems[b]).wait()

    def out_slice(c):
        h, j = chunk_hj(c)
        return out_hbm.at[h, :, pl.ds((col0 + j * _CHUNK), _CHUNK)]

    def write(c, b):
        pltpu.async_copy(t_v.at[b], out_slice(c), wsems[b])

    def wait_write(c, b):
        pltpu.make_async_copy(t_v.at[b], out_slice(c), wsems[b]).wait()

    def transpose(b):
        def trow(e):
            cid = jnp.full((_L,), e, jnp.int32)
            for n0 in range(_CHUNK // _L):
                v = plsc.load_gather(rows_v.at[b], [riota[n0], cid])
                t_v[b, e, pl.ds(n0 * _L, _L)] = v

        pl.loop(0, _EMB_DIM, unroll=2)(trow)

    for b in range(_LEAD):
        gather(b, b)

    for b in range(_NBUF):
        wait_gather(b, b)
        gather(b + _LEAD, (b + _LEAD) % _NBUF)
        transpose(b)
        write(b, b)

    def step(c0):
        for b in range(_NBUF):
            c = c0 + b
            wait_write(c - _NBUF, b)
            wait_gather(c, b)
            gather(c + _LEAD, (b + _LEAD) % _NBUF)
            transpose(b)
            write(c, b)

    pl.loop(_NBUF, _NCH - _NBUF, step=_NBUF)(step)

    for b in range(_NBUF):
        c = _NCH - _NBUF + b
        wait_write(c - _NBUF, b)
        wait_gather(c, b)
        if b < _NBUF - _LEAD:
            gather(c + _LEAD, (b + _LEAD) % _NBUF)
        transpose(b)
        write(c, b)
    for b in range(_NBUF):
        wait_write(_NCH - _NBUF + b, b)


@functools.partial(jax.jit, donate_argnums=())
def kernel(x, weight):
    xt = x.T
    out = pl.kernel(
        _emb_body,
        out_type=jax.ShapeDtypeStruct((_HIST_LEN, _EMB_DIM, _BATCH), jnp.float32),
        mesh=plsc.VectorSubcoreMesh(core_axis_name="c", subcore_axis_name="s"),
        scratch_types=[
            pltpu.VMEM((_HIST_LEN, _N_PER_W), jnp.int32),
            pltpu.VMEM((_NBUF, _CHUNK, _EMB_DIM), jnp.float32),
            pltpu.VMEM((_NBUF, _EMB_DIM, _CHUNK), jnp.float32),
        ] + [pltpu.SemaphoreType.DMA] * (2 * _NBUF),
        compiler_params=pltpu.CompilerParams(
            use_tc_tiling_on_sc=False, needs_layout_passes=False),
    )(xt, weight)
    return out.transpose(2, 0, 1)


# speedup vs baseline: 1.6560x; 1.6560x over previous
import functools

import jax
import jax.numpy as jnp
from jax import lax
from jax.experimental import pallas as pl
from jax.experimental.pallas import tpu as pltpu
from jax.experimental.pallas import tpu_sc as plsc

_VOCAB = 1_000_000
_EMB_DIM = 64
_BATCH = 16384
_HIST_LEN = 50

_NC = 2
_NS = 16
_NW = _NC * _NS
_L = 16

_N_PER_W = _BATCH // _NW
_CHUNK = 128
_JBLK = _N_PER_W // _CHUNK
_NCH = _HIST_LEN * _JBLK

_NBUF = 4
_LEAD = 2


def _emb_body(xt_hbm, w_hbm, out_hbm, idx_v, rows_v, t_v, *sems):
    gsems, wsems = sems[:_NBUF], sems[_NBUF:]
    wid = lax.axis_index("s") * _NC + lax.axis_index("c")
    col0 = wid * _N_PER_W
    pltpu.sync_copy(xt_hbm.at[:, pl.ds(col0, _N_PER_W)], idx_v)

    lane = lax.iota(jnp.int32, _L)
    eiota = [lane + e0 * _L for e0 in range(_EMB_DIM // _L)]

    def chunk_hj(c):
        return c % _HIST_LEN, c // _HIST_LEN

    def gather(c, b):
        h, j = chunk_hj(c)
        pltpu.async_copy(w_hbm.at[idx_v.at[h, pl.ds(j * _CHUNK, _CHUNK)]],
                         rows_v.at[b], gsems[b])

    def wait_gather(c, b):
        h, j = chunk_hj(c)
        pltpu.make_async_copy(
            w_hbm.at[idx_v.at[h, pl.ds(j * _CHUNK, _CHUNK)]],
            rows_v.at[b], gsems[b]).wait()

    def out_slice(c):
        h, j = chunk_hj(c)
        return out_hbm.at[h, :, pl.ds((col0 + j * _CHUNK), _CHUNK)]

    def write(c, b):
        pltpu.async_copy(t_v.at[b, :, pl.ds(0, _CHUNK)], out_slice(c),
                         wsems[b])

    def wait_write(c, b):
        pltpu.make_async_copy(t_v.at[b, :, pl.ds(0, _CHUNK)], out_slice(c),
                              wsems[b]).wait()

    def transpose(b):
        def trow(r):
            rid = jnp.full((_L,), r, jnp.int32)
            for e0 in range(_EMB_DIM // _L):
                v = rows_v[b, r, pl.ds(e0 * _L, _L)]
                plsc.store_scatter(t_v.at[b], [eiota[e0], rid], v)

        pl.loop(0, _CHUNK, unroll=2)(trow)

    for b in range(_LEAD):
        gather(b, b)

    for b in range(_NBUF):
        wait_gather(b, b)
        gather(b + _LEAD, (b + _LEAD) % _NBUF)
        transpose(b)
        write(b, b)

    def step(c0):
        for b in range(_NBUF):
            c = c0 + b
            wait_write(c - _NBUF, b)
            wait_gather(c, b)
            gather(c + _LEAD, (b + _LEAD) % _NBUF)
            transpose(b)
            write(c, b)

    pl.loop(_NBUF, _NCH - _NBUF, step=_NBUF)(step)

    for b in range(_NBUF):
        c = _NCH - _NBUF + b
        wait_write(c - _NBUF, b)
        wait_gather(c, b)
        if b < _NBUF - _LEAD:
            gather(c + _LEAD, (b + _LEAD) % _NBUF)
        transpose(b)
        write(c, b)
    for b in range(_NBUF):
        wait_write(_NCH - _NBUF + b, b)


@functools.partial(jax.jit, donate_argnums=())
def kernel(x, weight):
    xt = x.T
    out = pl.kernel(
        _emb_body,
        out_type=jax.ShapeDtypeStruct((_HIST_LEN, _EMB_DIM, _BATCH), jnp.float32),
        mesh=plsc.VectorSubcoreMesh(core_axis_name="c", subcore_axis_name="s"),
        scratch_types=[
            pltpu.VMEM((_HIST_LEN, _N_PER_W), jnp.int32),
            pltpu.VMEM((_NBUF, _CHUNK, _EMB_DIM), jnp.float32),
            pltpu.VMEM((_NBUF, _EMB_DIM, _CHUNK + 5), jnp.float32),
        ] + [pltpu.SemaphoreType.DMA] * (2 * _NBUF),
        compiler_params=pltpu.CompilerParams(
            use_tc_tiling_on_sc=False, needs_layout_passes=False),
    )(xt, weight)
    return out.transpose(2, 0, 1)


# speedup vs baseline: 1.6639x; 1.0048x over previous
import functools

import jax
import jax.numpy as jnp
from jax import lax
from jax.experimental import pallas as pl
from jax.experimental.pallas import tpu as pltpu
from jax.experimental.pallas import tpu_sc as plsc

_VOCAB = 1_000_000
_EMB_DIM = 64
_BATCH = 16384
_HIST_LEN = 50

_NC = 2
_NS = 16
_NW = _NC * _NS
_L = 16

_N_PER_W = _BATCH // _NW
_CHUNK = 128
_JBLK = _N_PER_W // _CHUNK
_NCH = _HIST_LEN * _JBLK

_NBUF = 4
_LEAD = 2


def _emb_body(xt_hbm, w_hbm, out_hbm, idx_v, rows_v, t_v, *sems):
    gsems, wsems = sems[:_NBUF], sems[_NBUF:]
    wid = lax.axis_index("s") * _NC + lax.axis_index("c")
    col0 = wid * _N_PER_W
    pltpu.sync_copy(xt_hbm.at[:, pl.ds(col0, _N_PER_W)], idx_v)

    lane = lax.iota(jnp.int32, _L)
    eiota = [lane + e0 * _L for e0 in range(_EMB_DIM // _L)]

    def chunk_hj(c):
        return c % _HIST_LEN, c // _HIST_LEN

    def gather(c, b):
        h, j = chunk_hj(c)
        pltpu.async_copy(w_hbm.at[idx_v.at[h, pl.ds(j * _CHUNK, _CHUNK)]],
                         rows_v.at[b], gsems[b])

    def wait_gather(c, b):
        h, j = chunk_hj(c)
        pltpu.make_async_copy(
            w_hbm.at[idx_v.at[h, pl.ds(j * _CHUNK, _CHUNK)]],
            rows_v.at[b], gsems[b]).wait()

    def out_slice(c):
        h, j = chunk_hj(c)
        return out_hbm.at[h, :, pl.ds((col0 + j * _CHUNK), _CHUNK)]

    def write(c, b):
        pltpu.async_copy(t_v.at[b, :, pl.ds(0, _CHUNK)], out_slice(c),
                         wsems[b])

    def wait_write(c, b):
        pltpu.make_async_copy(t_v.at[b, :, pl.ds(0, _CHUNK)], out_slice(c),
                              wsems[b]).wait()

    def transpose(b):
        def trow(r):
            rid = jnp.full((_L,), r, jnp.int32)
            for e0 in range(_EMB_DIM // _L):
                v = rows_v[b, r, pl.ds(e0 * _L, _L)]
                plsc.store_scatter(t_v.at[b], [eiota[e0], rid], v)

        pl.loop(0, _CHUNK, unroll=4)(trow)

    for b in range(_LEAD):
        gather(b, b)

    for b in range(_NBUF):
        wait_gather(b, b)
        gather(b + _LEAD, (b + _LEAD) % _NBUF)
        transpose(b)
        write(b, b)

    def step(c0):
        for b in range(_NBUF):
            c = c0 + b
            wait_write(c - _NBUF, b)
            wait_gather(c, b)
            gather(c + _LEAD, (b + _LEAD) % _NBUF)
            transpose(b)
            write(c, b)

    pl.loop(_NBUF, _NCH - _NBUF, step=_NBUF)(step)

    for b in range(_NBUF):
        c = _NCH - _NBUF + b
        wait_write(c - _NBUF, b)
        wait_gather(c, b)
        if b < _NBUF - _LEAD:
            gather(c + _LEAD, (b + _LEAD) % _NBUF)
        transpose(b)
        write(c, b)
    for b in range(_NBUF):
        wait_write(_NCH - _NBUF + b, b)


@functools.partial(jax.jit, donate_argnums=())
def kernel(x, weight):
    xt = x.T
    out = pl.kernel(
        _emb_body,
        out_type=jax.ShapeDtypeStruct((_HIST_LEN, _EMB_DIM, _BATCH), jnp.float32),
        mesh=plsc.VectorSubcoreMesh(core_axis_name="c", subcore_axis_name="s"),
        scratch_types=[
            pltpu.VMEM((_HIST_LEN, _N_PER_W), jnp.int32),
            pltpu.VMEM((_NBUF, _CHUNK, _EMB_DIM), jnp.float32),
            pltpu.VMEM((_NBUF, _EMB_DIM, _CHUNK + 5), jnp.float32),
        ] + [pltpu.SemaphoreType.DMA] * (2 * _NBUF),
        compiler_params=pltpu.CompilerParams(
            use_tc_tiling_on_sc=False, needs_layout_passes=False),
    )(xt, weight)
    return out.transpose(2, 0, 1)


# speedup vs baseline: 1.6700x; 1.0037x over previous
import functools

import jax
import jax.numpy as jnp
from jax import lax
from jax.experimental import pallas as pl
from jax.experimental.pallas import tpu as pltpu
from jax.experimental.pallas import tpu_sc as plsc

_VOCAB = 1_000_000
_EMB_DIM = 64
_BATCH = 16384
_HIST_LEN = 50

_NC = 2
_NS = 16
_NW = _NC * _NS
_L = 16

_N_PER_W = _BATCH // _NW
_CHUNK = 128
_JBLK = _N_PER_W // _CHUNK
_NCH = _HIST_LEN * _JBLK

_NBUF = 4
_LEAD = 2


def _emb_body(xt_hbm, w_hbm, out_hbm, idx_v, rows_v, t_v, *sems):
    gsems, wsems = sems[:_NBUF], sems[_NBUF:]
    wid = lax.axis_index("s") * _NC + lax.axis_index("c")
    col0 = wid * _N_PER_W
    pltpu.sync_copy(xt_hbm.at[:, pl.ds(col0, _N_PER_W)], idx_v)

    lane = lax.iota(jnp.int32, _L)
    eiota = [lane + e0 * _L for e0 in range(_EMB_DIM // _L)]

    def chunk_hj(c):
        return c % _HIST_LEN, c // _HIST_LEN

    def gather(c, b):
        h, j = chunk_hj(c)
        pltpu.async_copy(w_hbm.at[idx_v.at[h, pl.ds(j * _CHUNK, _CHUNK)]],
                         rows_v.at[b], gsems[b])

    def wait_gather(c, b):
        h, j = chunk_hj(c)
        pltpu.make_async_copy(
            w_hbm.at[idx_v.at[h, pl.ds(j * _CHUNK, _CHUNK)]],
            rows_v.at[b], gsems[b]).wait()

    def out_slice(c):
        h, j = chunk_hj(c)
        return out_hbm.at[h, :, pl.ds((col0 + j * _CHUNK), _CHUNK)]

    def write(c, b):
        pltpu.async_copy(t_v.at[b, :, pl.ds(0, _CHUNK)], out_slice(c),
                         wsems[b])

    def wait_write(c, b):
        pltpu.make_async_copy(t_v.at[b, :, pl.ds(0, _CHUNK)], out_slice(c),
                              wsems[b]).wait()

    def transpose(b):
        def trow(r):
            rid = jnp.full((_L,), r, jnp.int32)
            for e0 in range(_EMB_DIM // _L):
                v = rows_v[b, r, pl.ds(e0 * _L, _L)]
                plsc.store_scatter(t_v.at[b], [eiota[e0], rid], v)

        pl.loop(0, _CHUNK, unroll=8)(trow)

    for b in range(_LEAD):
        gather(b, b)

    for b in range(_NBUF):
        wait_gather(b, b)
        gather(b + _LEAD, (b + _LEAD) % _NBUF)
        transpose(b)
        write(b, b)

    def step(c0):
        for b in range(_NBUF):
            c = c0 + b
            wait_write(c - _NBUF, b)
            wait_gather(c, b)
            gather(c + _LEAD, (b + _LEAD) % _NBUF)
            transpose(b)
            write(c, b)

    pl.loop(_NBUF, _NCH - _NBUF, step=_NBUF)(step)

    for b in range(_NBUF):
        c = _NCH - _NBUF + b
        wait_write(c - _NBUF, b)
        wait_gather(c, b)
        if b < _NBUF - _LEAD:
            gather(c + _LEAD, (b + _LEAD) % _NBUF)
        transpose(b)
        write(c, b)
    for b in range(_NBUF):
        wait_write(_NCH - _NBUF + b, b)


@functools.partial(jax.jit, donate_argnums=())
def kernel(x, weight):
    xt = x.T
    out = pl.kernel(
        _emb_body,
        out_type=jax.ShapeDtypeStruct((_HIST_LEN, _EMB_DIM, _BATCH), jnp.float32),
        mesh=plsc.VectorSubcoreMesh(core_axis_name="c", subcore_axis_name="s"),
        scratch_types=[
            pltpu.VMEM((_HIST_LEN, _N_PER_W), jnp.int32),
            pltpu.VMEM((_NBUF, _CHUNK, _EMB_DIM), jnp.float32),
            pltpu.VMEM((_NBUF, _EMB_DIM, _CHUNK + 5), jnp.float32),
        ] + [pltpu.SemaphoreType.DMA] * (2 * _NBUF),
        compiler_params=pltpu.CompilerParams(
            use_tc_tiling_on_sc=False, needs_layout_passes=False),
    )(xt, weight)
    return out.transpose(2, 0, 1)


# speedup vs baseline: 1.8072x; 1.0821x over previous
import functools

import jax
import jax.numpy as jnp
from jax import lax
from jax.experimental import pallas as pl
from jax.experimental.pallas import tpu as pltpu
from jax.experimental.pallas import tpu_sc as plsc

_VOCAB = 1_000_000
_EMB_DIM = 64
_BATCH = 16384
_HIST_LEN = 50

_NC = 2
_NS = 16
_NW = _NC * _NS
_L = 16

_N_PER_W = _BATCH // _NW
_CHUNK = 128
_JBLK = _N_PER_W // _CHUNK
_NCH = _HIST_LEN * _JBLK

_NBUF = 4
_LEAD = 2


def _emb_body(xt_hbm, w_hbm, out_hbm, idx_v, rows_v, t_v, *sems):
    gsems, wsems = sems[:_NBUF], sems[_NBUF:]
    wid = lax.axis_index("s") * _NC + lax.axis_index("c")
    col0 = wid * _N_PER_W
    pltpu.sync_copy(xt_hbm.at[:, pl.ds(col0, _N_PER_W)], idx_v)

    lane = lax.iota(jnp.int32, _L)
    eiota = [lane + e0 * _L for e0 in range(_EMB_DIM // _L)]

    def chunk_hj(c):
        return c % _HIST_LEN, c // _HIST_LEN

    def gather(c, b):
        h, j = chunk_hj(c)
        pltpu.async_copy(w_hbm.at[idx_v.at[h, pl.ds(j * _CHUNK, _CHUNK)]],
                         rows_v.at[b], gsems[b])

    def wait_gather(c, b):
        h, j = chunk_hj(c)
        pltpu.make_async_copy(
            w_hbm.at[idx_v.at[h, pl.ds(j * _CHUNK, _CHUNK)]],
            rows_v.at[b], gsems[b]).wait()

    def out_slice(c):
        h, j = chunk_hj(c)
        return out_hbm.at[h, :, pl.ds((col0 + j * _CHUNK), _CHUNK)]

    def write(c, b):
        pltpu.async_copy(t_v.at[b, :, pl.ds(0, _CHUNK)], out_slice(c),
                         wsems[b])

    def wait_write(c, b):
        pltpu.make_async_copy(t_v.at[b, :, pl.ds(0, _CHUNK)], out_slice(c),
                              wsems[b]).wait()

    ones = jnp.full((_L,), 1, jnp.int32)

    def transpose(b):
        def tgroup(g):
            rid = jnp.full((_L,), g * _L, jnp.int32)
            for i in range(_L):
                r = g * _L + i
                vs = [rows_v[b, r, pl.ds(e0 * _L, _L)]
                      for e0 in range(_EMB_DIM // _L)]
                for e0 in range(_EMB_DIM // _L):
                    plsc.store_scatter(t_v.at[b], [eiota[e0], rid], vs[e0])
                rid = rid + ones

        pl.loop(0, _CHUNK // _L)(tgroup)

    for b in range(_LEAD):
        gather(b, b)

    for b in range(_NBUF):
        wait_gather(b, b)
        gather(b + _LEAD, (b + _LEAD) % _NBUF)
        transpose(b)
        write(b, b)

    def step(c0):
        for b in range(_NBUF):
            c = c0 + b
            wait_write(c - _NBUF, b)
            wait_gather(c, b)
            gather(c + _LEAD, (b + _LEAD) % _NBUF)
            transpose(b)
            write(c, b)

    pl.loop(_NBUF, _NCH - _NBUF, step=_NBUF)(step)

    for b in range(_NBUF):
        c = _NCH - _NBUF + b
        wait_write(c - _NBUF, b)
        wait_gather(c, b)
        if b < _NBUF - _LEAD:
            gather(c + _LEAD, (b + _LEAD) % _NBUF)
        transpose(b)
        write(c, b)
    for b in range(_NBUF):
        wait_write(_NCH - _NBUF + b, b)


@functools.partial(jax.jit, donate_argnums=())
def kernel(x, weight):
    xt = x.T
    out = pl.kernel(
        _emb_body,
        out_type=jax.ShapeDtypeStruct((_HIST_LEN, _EMB_DIM, _BATCH), jnp.float32),
        mesh=plsc.VectorSubcoreMesh(core_axis_name="c", subcore_axis_name="s"),
        scratch_types=[
            pltpu.VMEM((_HIST_LEN, _N_PER_W), jnp.int32),
            pltpu.VMEM((_NBUF, _CHUNK, _EMB_DIM), jnp.float32),
            pltpu.VMEM((_NBUF, _EMB_DIM, _CHUNK + 5), jnp.float32),
        ] + [pltpu.SemaphoreType.DMA] * (2 * _NBUF),
        compiler_params=pltpu.CompilerParams(
            use_tc_tiling_on_sc=False, needs_layout_passes=False),
    )(xt, weight)
    return out.transpose(2, 0, 1)


# speedup vs baseline: 2.2628x; 1.2521x over previous
import functools

import jax
import jax.numpy as jnp
from jax import lax
from jax.experimental import pallas as pl
from jax.experimental.pallas import tpu as pltpu
from jax.experimental.pallas import tpu_sc as plsc

_VOCAB = 1_000_000
_EMB_DIM = 64
_BATCH = 16384
_HIST_LEN = 50

_NC = 2
_NS = 16
_NW = _NC * _NS
_L = 16

_N_PER_W = _BATCH // _NW
_CHUNK = 128
_JBLK = _N_PER_W // _CHUNK
_NCH = _HIST_LEN * _JBLK

_NBUF = 4
_LEAD = 2


def _emb_body(xt_hbm, w_hbm, out_hbm, idx_v, rows_v, t_v, *sems):
    gsems, wsems = sems[:_NBUF], sems[_NBUF:]
    wid = lax.axis_index("s") * _NC + lax.axis_index("c")
    col0 = wid * _N_PER_W
    pltpu.sync_copy(xt_hbm.at[:, pl.ds(col0, _N_PER_W)], idx_v)

    lane = lax.iota(jnp.int32, _L)
    eiota = [lane + e0 * _L for e0 in range(_EMB_DIM // _L)]
    ehi = [e // 8 for e in eiota]
    elo = [e % 8 for e in eiota]

    def chunk_hj(c):
        return c % _HIST_LEN, c // _HIST_LEN

    def gather(c, b):
        h, j = chunk_hj(c)
        pltpu.async_copy(w_hbm.at[idx_v.at[h, pl.ds(j * _CHUNK, _CHUNK)]],
                         rows_v.at[b], gsems[b])

    def wait_gather(c, b):
        h, j = chunk_hj(c)
        pltpu.make_async_copy(
            w_hbm.at[idx_v.at[h, pl.ds(j * _CHUNK, _CHUNK)]],
            rows_v.at[b], gsems[b]).wait()

    def out_slice(c):
        h, j = chunk_hj(c)
        return out_hbm.at[h, :, wid * _JBLK + j]

    def write(c, b):
        pltpu.async_copy(t_v.at[b, :, :, pl.ds(0, _CHUNK)], out_slice(c),
                         wsems[b])

    def wait_write(c, b):
        pltpu.make_async_copy(t_v.at[b, :, :, pl.ds(0, _CHUNK)], out_slice(c),
                              wsems[b]).wait()

    ones = jnp.full((_L,), 1, jnp.int32)

    def transpose(b):
        def tgroup(g):
            rid = jnp.full((_L,), g * _L, jnp.int32)
            for i in range(_L):
                r = g * _L + i
                vs = [rows_v[b, r, pl.ds(e0 * _L, _L)]
                      for e0 in range(_EMB_DIM // _L)]
                for e0 in range(_EMB_DIM // _L):
                    plsc.store_scatter(t_v.at[b], [ehi[e0], elo[e0], rid], vs[e0])
                rid = rid + ones

        pl.loop(0, _CHUNK // _L)(tgroup)

    for b in range(_LEAD):
        gather(b, b)

    for b in range(_NBUF):
        wait_gather(b, b)
        gather(b + _LEAD, (b + _LEAD) % _NBUF)
        transpose(b)
        write(b, b)

    def step(c0):
        for b in range(_NBUF):
            c = c0 + b
            wait_write(c - _NBUF, b)
            wait_gather(c, b)
            gather(c + _LEAD, (b + _LEAD) % _NBUF)
            transpose(b)
            write(c, b)

    pl.loop(_NBUF, _NCH - _NBUF, step=_NBUF)(step)

    for b in range(_NBUF):
        c = _NCH - _NBUF + b
        wait_write(c - _NBUF, b)
        wait_gather(c, b)
        if b < _NBUF - _LEAD:
            gather(c + _LEAD, (b + _LEAD) % _NBUF)
        transpose(b)
        write(c, b)
    for b in range(_NBUF):
        wait_write(_NCH - _NBUF + b, b)


@functools.partial(jax.jit, donate_argnums=())
def kernel(x, weight):
    xt = x.T
    out = pl.kernel(
        _emb_body,
        out_type=jax.ShapeDtypeStruct(
            (_HIST_LEN, 8, _BATCH // _CHUNK, 8, _CHUNK), jnp.float32),
        mesh=plsc.VectorSubcoreMesh(core_axis_name="c", subcore_axis_name="s"),
        scratch_types=[
            pltpu.VMEM((_HIST_LEN, _N_PER_W), jnp.int32),
            pltpu.VMEM((_NBUF, _CHUNK, _EMB_DIM), jnp.float32),
            pltpu.VMEM((_NBUF, 8, 8, _CHUNK + 5), jnp.float32),
        ] + [pltpu.SemaphoreType.DMA] * (2 * _NBUF),
        compiler_params=pltpu.CompilerParams(
            use_tc_tiling_on_sc=False, needs_layout_passes=False),
    )(xt, weight)
    return out.transpose(2, 4, 0, 1, 3).reshape(_BATCH, _HIST_LEN, _EMB_DIM)
